# baseline (speedup 1.0000x reference)
import jax
import jax.numpy as jnp
from jax import lax
from jax.experimental import pallas as pl
from jax.experimental.pallas import tpu as pltpu
from jax.experimental.pallas import tpu_sc as plsc

_D = 64
_BATCH = 16384
_NEG_N = 100
_MARGIN = 0.8
_NITEM = 1000000

_NC, _NS, _L = 2, 16, 16
_NW = _NC * _NS
_EPW = _BATCH // _NW
_C = 8
_NCHUNK = _EPW // _C
_NPC = _C * _NEG_N
_G = -(-_NEG_N // _L)


def _sc_body(u_hbm, i_hbm, n_hbm, user_hbm, item_hbm, out_hbm,
             uix0, uix1, iix0, iix1, nix0, nix1,
             ue0, ue1, ie0, ie1, nb0, nb1, acc_v,
             gsem0, gsem1, isem0, isem1):
  wid = lax.axis_index("s") * _NC + lax.axis_index("c")
  base = wid * _EPW
  iota = lax.iota(jnp.int32, _L)
  rows = [jnp.minimum(g * _L + iota, _NEG_N - 1) for g in range(_G)]
  lanepos = [g * _L + iota for g in range(_G)]
  prow = jnp.minimum(iota, _C - 1)

  bufs = ((uix0, iix0, nix0, ue0, ie0, nb0, gsem0, isem0),
          (uix1, iix1, nix1, ue1, ie1, nb1, gsem1, isem1))

  def stage_idx(chunk, buf):
    uix, iix, nix, _, _, _, _, isem = bufs[buf]
    e0 = base + chunk * _C
    pltpu.async_copy(u_hbm.at[pl.ds(e0, _C)], uix.at[pl.ds(0, _C)], isem)
    pltpu.async_copy(i_hbm.at[pl.ds(e0, _C)], iix.at[pl.ds(0, _C)], isem)
    pltpu.async_copy(n_hbm.at[pl.ds(e0 * _NEG_N, _NPC)], nix, isem)

  def wait_idx(buf):
    uix, iix, nix, _, _, _, _, isem = bufs[buf]
    pltpu.make_async_copy(u_hbm.at[pl.ds(0, _C)], uix.at[pl.ds(0, _C)],
                          isem).wait()
    pltpu.make_async_copy(i_hbm.at[pl.ds(0, _C)], iix.at[pl.ds(0, _C)],
                          isem).wait()
    pltpu.make_async_copy(n_hbm.at[pl.ds(0, _NPC)], nix, isem).wait()

  def _xf(k):
    return (k & -1024) + ((k & 511) << 1) + ((k >> 9) & 1)

  def xform_idx(buf):
    uix, iix, nix, _, _, _, _, _ = bufs[buf]
    uix[...] = _xf(uix[...])
    iix[...] = _xf(iix[...])
    for v in range(_NPC // _L):
      nix[pl.ds(v * _L, _L)] = _xf(nix[pl.ds(v * _L, _L)])

  def issue_gathers(buf):
    uix, iix, nix, ue, ie, nb, gsem, _ = bufs[buf]
    pltpu.async_copy(user_hbm.at[uix.at[pl.ds(0, _C)]], ue, gsem)
    pltpu.async_copy(item_hbm.at[iix.at[pl.ds(0, _C)]], ie, gsem)
    for j in range(10):
      pltpu.async_copy(item_hbm.at[nix.at[pl.ds(j * 80, 80)]],
                       nb.at[pl.ds(j * 80, 80)], gsem)

  def wait_gathers(buf):
    uix, iix, nix, ue, ie, nb, gsem, _ = bufs[buf]
    pltpu.make_async_copy(user_hbm.at[uix.at[pl.ds(0, _C)]], ue, gsem).wait()
    pltpu.make_async_copy(item_hbm.at[iix.at[pl.ds(0, _C)]], ie, gsem).wait()
    for j in range(10):
      pltpu.make_async_copy(item_hbm.at[nix.at[pl.ds(j * 80, 80)]],
                            nb.at[pl.ds(j * 80, 80)], gsem).wait()

  def compute(buf, acc):
    _, _, _, ue, ie, nb, _, _ = bufs[buf]

    def pstep(d, p):
      dvec = (d + iota) & (_D - 1)
      return p + (plsc.load_gather(ue, [prow, dvec]) *
                  plsc.load_gather(ie, [prow, dvec]))

    ppred = lax.fori_loop(0, _D, pstep, jnp.zeros((_L,), jnp.float32),
                          unroll=4)
    acc = acc + jnp.where(iota < _C, jnp.maximum(1.0 - ppred, 0.0), 0.0)

    for b in range(_C):
      nbb = nb.at[pl.ds(b * _NEG_N, _NEG_N)]
      bsplat = jnp.full((_L,), b, jnp.int32)

      def dstep(d, ps):
        dvec = (d + iota) & (_D - 1)
        uv = plsc.load_gather(ue, [bsplat, dvec])
        return tuple(ps[g] + uv * plsc.load_gather(nbb, [rows[g], dvec])
                     for g in range(_G))

      preds = lax.fori_loop(
          0, _D, dstep,
          tuple(jnp.zeros((_L,), jnp.float32) for _ in range(_G)),
          unroll=4)
      for g in range(_G):
        neg = jnp.maximum(preds[g] - _MARGIN, 0.0)
        acc = acc + jnp.where(lanepos[g] < _NEG_N, neg, 0.0)
    return acc

  stage_idx(0, 0)
  stage_idx(1, 1)
  wait_idx(0)
  xform_idx(0)
  issue_gathers(0)
  wait_idx(1)
  xform_idx(1)
  issue_gathers(1)

  def outer(t, acc):
    for buf in range(2):
      chunk = 2 * t + buf
      wait_gathers(buf)

      @pl.when(chunk + 2 < _NCHUNK)
      def _():
        stage_idx(chunk + 2, buf)

      acc = compute(buf, acc)

      @pl.when(chunk + 2 < _NCHUNK)
      def _():
        wait_idx(buf)
        xform_idx(buf)
        issue_gathers(buf)
    return acc

  acc = lax.fori_loop(0, _NCHUNK // 2, outer, jnp.zeros((_L,), jnp.float32))
  acc_v[...] = acc
  pltpu.sync_copy(acc_v, out_hbm.at[pl.ds(wid * _L, _L)])


_sc_call = pl.kernel(
    _sc_body,
    out_type=jax.ShapeDtypeStruct((_NW * _L,), jnp.float32),
    mesh=plsc.VectorSubcoreMesh(core_axis_name="c", subcore_axis_name="s",
                                num_cores=_NC, num_subcores=_NS),
    compiler_params=pltpu.CompilerParams(needs_layout_passes=False,
                                         use_tc_tiling_on_sc=False),
    scratch_types=[
        pltpu.VMEM((_L,), jnp.int32),
        pltpu.VMEM((_L,), jnp.int32),
        pltpu.VMEM((_L,), jnp.int32),
        pltpu.VMEM((_L,), jnp.int32),
        pltpu.VMEM((_NPC,), jnp.int32),
        pltpu.VMEM((_NPC,), jnp.int32),
        pltpu.VMEM((_C, _D), jnp.float32),
        pltpu.VMEM((_C, _D), jnp.float32),
        pltpu.VMEM((_C, _D), jnp.float32),
        pltpu.VMEM((_C, _D), jnp.float32),
        pltpu.VMEM((_NPC, _D), jnp.float32),
        pltpu.VMEM((_NPC, _D), jnp.float32),
        pltpu.VMEM((_L,), jnp.float32),
        pltpu.SemaphoreType.DMA,
        pltpu.SemaphoreType.DMA,
        pltpu.SemaphoreType.DMA,
        pltpu.SemaphoreType.DMA,
    ],
)


_RB = 1024


def _detile_body(src_ref, out_ref):
  top = src_ref[0:_RB // 2, :]
  bot = src_ref[_RB // 2:_RB, :]
  out_ref[...] = jnp.concatenate([top, bot], axis=1).reshape(-1)


def _detile_tc(table, nrows_used):
  nb = -(-nrows_used // _RB)
  return pl.pallas_call(
      _detile_body,
      grid=(nb,),
      in_specs=[pl.BlockSpec((_RB, _D), lambda j: (j, 0))],
      out_specs=pl.BlockSpec((_RB * _D,), lambda j: (j,)),
      out_shape=jax.ShapeDtypeStruct((nb * _RB * _D,), jnp.float32),
  )(table)


@jax.jit
def kernel(u, i, user_table, item_table):
  neg = jax.random.randint(jax.random.key(42), (_BATCH * _NEG_N,), 0, _NITEM)
  uflat = _detile_tc(user_table, user_table.shape[0] - 1)
  iflat = _detile_tc(item_table, item_table.shape[0] - 1)
  out = _sc_call(u.astype(jnp.int32), i.astype(jnp.int32),
                 neg.astype(jnp.int32),
                 uflat.reshape(-1, _D),
                 iflat.reshape(-1, _D))
  return jnp.sum(out) / _BATCH


# speedup vs baseline: 1.6246x; 1.6246x over previous
import jax
import jax.numpy as jnp
from jax import lax
from jax.experimental import pallas as pl
from jax.experimental.pallas import tpu as pltpu
from jax.experimental.pallas import tpu_sc as plsc

_D = 64
_BATCH = 16384
_NEG_N = 100
_MARGIN = 0.8
_NITEM = 1000000

_NC, _NS, _L = 2, 16, 16
_NW = _NC * _NS
_EPW = _BATCH // _NW
_C = 8
_NCHUNK = _EPW // _C
_NPC = _C * _NEG_N
_G = -(-_NEG_N // _L)


def _sc_body(u_hbm, i_hbm, n_hbm, user_hbm, item_hbm, out_hbm,
             uix0, uix1, iix0, iix1, nix0, nix1,
             ue0, ue1, ie0, ie1, nb0, nb1, acc_v,
             gsem0, gsem1, isem0, isem1):
  wid = lax.axis_index("s") * _NC + lax.axis_index("c")
  base = wid * _EPW
  iota = lax.iota(jnp.int32, _L)
  rows = [jnp.minimum(g * _L + iota, _NEG_N - 1) for g in range(_G)]
  lanepos = [g * _L + iota for g in range(_G)]
  prow = jnp.minimum(iota, _C - 1)

  bufs = ((uix0, iix0, nix0, ue0, ie0, nb0, gsem0, isem0),
          (uix1, iix1, nix1, ue1, ie1, nb1, gsem1, isem1))

  def stage_idx(chunk, buf):
    uix, iix, nix, _, _, _, _, isem = bufs[buf]
    e0 = base + chunk * _C
    pltpu.async_copy(u_hbm.at[pl.ds(e0, _C)], uix.at[pl.ds(0, _C)], isem)
    pltpu.async_copy(i_hbm.at[pl.ds(e0, _C)], iix.at[pl.ds(0, _C)], isem)
    pltpu.async_copy(n_hbm.at[pl.ds(e0 * _NEG_N, _NPC)], nix, isem)

  def wait_idx(buf):
    uix, iix, nix, _, _, _, _, isem = bufs[buf]
    pltpu.make_async_copy(u_hbm.at[pl.ds(0, _C)], uix.at[pl.ds(0, _C)],
                          isem).wait()
    pltpu.make_async_copy(i_hbm.at[pl.ds(0, _C)], iix.at[pl.ds(0, _C)],
                          isem).wait()
    pltpu.make_async_copy(n_hbm.at[pl.ds(0, _NPC)], nix, isem).wait()

  def issue_gathers(buf):
    uix, iix, nix, ue, ie, nb, gsem, _ = bufs[buf]
    pltpu.async_copy(user_hbm.at[uix.at[pl.ds(0, _C)]], ue, gsem)
    pltpu.async_copy(item_hbm.at[iix.at[pl.ds(0, _C)]], ie, gsem)
    for j in range(10):
      pltpu.async_copy(item_hbm.at[nix.at[pl.ds(j * 80, 80)]],
                       nb.at[pl.ds(j * 80, 80)], gsem)

  def wait_gathers(buf):
    uix, iix, nix, ue, ie, nb, gsem, _ = bufs[buf]
    pltpu.make_async_copy(user_hbm.at[uix.at[pl.ds(0, _C)]], ue, gsem).wait()
    pltpu.make_async_copy(item_hbm.at[iix.at[pl.ds(0, _C)]], ie, gsem).wait()
    for j in range(10):
      pltpu.make_async_copy(item_hbm.at[nix.at[pl.ds(j * 80, 80)]],
                            nb.at[pl.ds(j * 80, 80)], gsem).wait()

  def compute(buf, acc):
    _, _, _, ue, ie, nb, _, _ = bufs[buf]

    def pstep(d, p):
      dvec = (d + iota) & (_D - 1)
      return p + (plsc.load_gather(ue, [prow, dvec]) *
                  plsc.load_gather(ie, [prow, dvec]))

    ppred = lax.fori_loop(0, _D, pstep, jnp.zeros((_L,), jnp.float32),
                          unroll=4)
    acc = acc + jnp.where(iota < _C, jnp.maximum(1.0 - ppred, 0.0), 0.0)

    for b in range(_C):
      nbb = nb.at[pl.ds(b * _NEG_N, _NEG_N)]
      bsplat = jnp.full((_L,), b, jnp.int32)

      def dstep(d, ps):
        dvec = (d + iota) & (_D - 1)
        uv = plsc.load_gather(ue, [bsplat, dvec])
        return tuple(ps[g] + uv * plsc.load_gather(nbb, [rows[g], dvec])
                     for g in range(_G))

      preds = lax.fori_loop(
          0, _D, dstep,
          tuple(jnp.zeros((_L,), jnp.float32) for _ in range(_G)),
          unroll=4)
      for g in range(_G):
        neg = jnp.maximum(preds[g] - _MARGIN, 0.0)
        acc = acc + jnp.where(lanepos[g] < _NEG_N, neg, 0.0)
    return acc

  stage_idx(0, 0)
  stage_idx(1, 1)
  wait_idx(0)
  issue_gathers(0)
  wait_idx(1)
  issue_gathers(1)

  def outer(t, acc):
    for buf in range(2):
      chunk = 2 * t + buf
      wait_gathers(buf)

      @pl.when(chunk + 2 < _NCHUNK)
      def _():
        stage_idx(chunk + 2, buf)

      acc = compute(buf, acc)

      @pl.when(chunk + 2 < _NCHUNK)
      def _():
        wait_idx(buf)
        issue_gathers(buf)
    return acc

  acc = lax.fori_loop(0, _NCHUNK // 2, outer, jnp.zeros((_L,), jnp.float32))
  acc_v[...] = acc
  pltpu.sync_copy(acc_v, out_hbm.at[pl.ds(wid * _L, _L)])


_sc_call = pl.kernel(
    _sc_body,
    out_type=jax.ShapeDtypeStruct((_NW * _L,), jnp.float32),
    mesh=plsc.VectorSubcoreMesh(core_axis_name="c", subcore_axis_name="s",
                                num_cores=_NC, num_subcores=_NS),
    compiler_params=pltpu.CompilerParams(needs_layout_passes=False,
                                         use_tc_tiling_on_sc=False),
    scratch_types=[
        pltpu.VMEM((_L,), jnp.int32),
        pltpu.VMEM((_L,), jnp.int32),
        pltpu.VMEM((_L,), jnp.int32),
        pltpu.VMEM((_L,), jnp.int32),
        pltpu.VMEM((_NPC,), jnp.int32),
        pltpu.VMEM((_NPC,), jnp.int32),
        pltpu.VMEM((_C, _D), jnp.float32),
        pltpu.VMEM((_C, _D), jnp.float32),
        pltpu.VMEM((_C, _D), jnp.float32),
        pltpu.VMEM((_C, _D), jnp.float32),
        pltpu.VMEM((_NPC, _D), jnp.float32),
        pltpu.VMEM((_NPC, _D), jnp.float32),
        pltpu.VMEM((_L,), jnp.float32),
        pltpu.SemaphoreType.DMA,
        pltpu.SemaphoreType.DMA,
        pltpu.SemaphoreType.DMA,
        pltpu.SemaphoreType.DMA,
    ],
)


@jax.jit
def kernel(u, i, user_table, item_table):
  neg = jax.random.randint(jax.random.key(42), (_BATCH * _NEG_N,), 0, _NITEM)
  uflat = jax.lax.optimization_barrier(user_table.reshape(-1))
  iflat = jax.lax.optimization_barrier(item_table.reshape(-1))
  out = _sc_call(u.astype(jnp.int32), i.astype(jnp.int32),
                 neg.astype(jnp.int32),
                 uflat.reshape(-1, _D),
                 iflat.reshape(-1, _D))
  return jnp.sum(out) / _BATCH
